# Initial kernel scaffold; baseline (speedup 1.0000x reference)
import jax
import jax.numpy as jnp
from jax.experimental import pallas as pl


def kernel(x, edge_index, W1l, b1l, W1r, b1r, att1, bias1, bn1_g, bn1_b, W2l, b2l, W2r, b2r, att2, bias2, bn2_g, bn2_b, W3l, b3l, W3r, b3r, att3, bias3):
    raise NotImplementedError("write your pallas kernel here")


# speedup vs baseline: 7.8021x; 7.8021x over previous
import functools

import jax
import jax.numpy as jnp
from jax.experimental import pallas as pl

N = 50000
E_RAW = 800000
E_TOT = E_RAW + N

EB = 2000
NB = 2000


def _proj_body(x_ref, w_ref, b_ref, o_ref):
    o_ref[...] = (
        jnp.dot(x_ref[...], w_ref[...], preferred_element_type=jnp.float32)
        + b_ref[...]
    )


def _proj(x, w, b):
    n, din = x.shape
    d = w.shape[1]
    grid = n // NB
    return pl.pallas_call(
        _proj_body,
        grid=(grid,),
        in_specs=[
            pl.BlockSpec((NB, din), lambda i: (i, 0)),
            pl.BlockSpec((din, d), lambda i: (0, 0)),
            pl.BlockSpec((1, d), lambda i: (0, 0)),
        ],
        out_specs=pl.BlockSpec((NB, d), lambda i: (i, 0)),
        out_shape=jax.ShapeDtypeStruct((n, d), jnp.float32),
    )(x, w, b.reshape(1, d))


def _alpha_body(gl_ref, gr_ref, att_ref, a_ref, m_ref, *, h):
    e = gl_ref[...] + gr_ref[...]
    e = jnp.where(e > 0, e, 0.2 * e)
    eb = e.shape[0]
    a = (e.reshape(eb, h, 16) * att_ref[...]).sum(-1)
    if h < 8:
        a = jnp.pad(a, ((0, 0), (0, 8 - h)))
    a_ref[...] = a
    m_ref[...] = jnp.max(a, axis=0, keepdims=True)[None]


def _alpha(gl, gr, att):
    h = att.shape[0]
    d = gl.shape[1]
    grid = E_TOT // EB
    return pl.pallas_call(
        functools.partial(_alpha_body, h=h),
        grid=(grid,),
        in_specs=[
            pl.BlockSpec((EB, d), lambda i: (i, 0)),
            pl.BlockSpec((EB, d), lambda i: (i, 0)),
            pl.BlockSpec((h, 16), lambda i: (0, 0)),
        ],
        out_specs=[
            pl.BlockSpec((EB, 8), lambda i: (i, 0)),
            pl.BlockSpec((1, 1, 8), lambda i: (i, 0, 0)),
        ],
        out_shape=[
            jax.ShapeDtypeStruct((E_TOT, 8), jnp.float32),
            jax.ShapeDtypeStruct((grid, 1, 8), jnp.float32),
        ],
    )(gl, gr, att)


def _msg_body(gl_ref, a_ref, k_ref, msg_ref, s_ref, *, h):
    s = jnp.exp(a_ref[...] - k_ref[...])
    sexp = jnp.repeat(s[:, :h], 16, axis=1)
    msg_ref[...] = gl_ref[...] * sexp
    s_ref[...] = jnp.pad(s, ((0, 0), (0, 8)))


def _msg(gl, alpha, kmax, h):
    d = gl.shape[1]
    grid = E_TOT // EB
    return pl.pallas_call(
        functools.partial(_msg_body, h=h),
        grid=(grid,),
        in_specs=[
            pl.BlockSpec((EB, d), lambda i: (i, 0)),
            pl.BlockSpec((EB, 8), lambda i: (i, 0)),
            pl.BlockSpec((1, 1), lambda i: (0, 0)),
        ],
        out_specs=[
            pl.BlockSpec((EB, d), lambda i: (i, 0)),
            pl.BlockSpec((EB, 16), lambda i: (i, 0)),
        ],
        out_shape=[
            jax.ShapeDtypeStruct((E_TOT, d), jnp.float32),
            jax.ShapeDtypeStruct((E_TOT, 16), jnp.float32),
        ],
    )(gl, alpha, kmax.reshape(1, 1))


def _final_body(acc_ref, den_ref, b_ref, g_ref, bb_ref, o_ref, *, h, bn, elu):
    den = den_ref[...][:, :h]
    dexp = jnp.repeat(den, 16, axis=1) + 1e-16
    o = acc_ref[...] / dexp + b_ref[...]
    if bn:
        o = o / jnp.sqrt(jnp.float32(1.0 + 1e-5)) * g_ref[...] + bb_ref[...]
    if elu:
        o = jnp.where(o > 0, o, jnp.exp(jnp.minimum(o, 0.0)) - 1.0)
    o_ref[...] = o


def _final(acc, den, bias, g, bb, h, bn, elu):
    d = acc.shape[1]
    grid = N // NB
    return pl.pallas_call(
        functools.partial(_final_body, h=h, bn=bn, elu=elu),
        grid=(grid,),
        in_specs=[
            pl.BlockSpec((NB, d), lambda i: (i, 0)),
            pl.BlockSpec((NB, 16), lambda i: (i, 0)),
            pl.BlockSpec((1, d), lambda i: (0, 0)),
            pl.BlockSpec((1, d), lambda i: (0, 0)),
            pl.BlockSpec((1, d), lambda i: (0, 0)),
        ],
        out_specs=pl.BlockSpec((NB, d), lambda i: (i, 0)),
        out_shape=jax.ShapeDtypeStruct((N, d), jnp.float32),
    )(acc, den, bias.reshape(1, d), g.reshape(1, d), bb.reshape(1, d))


def _gather_rows(table, idx):
    return table[idx]


def _scatter_add_rows(vals, idx, n):
    return jax.ops.segment_sum(vals, idx, num_segments=n)


def _gat_layer(x, src, dst, wl, bl, wr, br, att, bias, h, bn_g, bn_b,
               use_bn, use_elu):
    d = wl.shape[1]
    xl = _proj(x, wl, bl)
    xr = _proj(x, wr, br)
    gl = _gather_rows(xl, src)
    gr = _gather_rows(xr, dst)
    alpha, bmax = _alpha(gl, gr, att)
    kmax = jnp.max(bmax)
    msg, spad = _msg(gl, alpha, kmax, h)
    acc = _scatter_add_rows(msg, dst, N)
    den = _scatter_add_rows(spad, dst, N)
    return _final(acc, den, bias, bn_g, bn_b, h, use_bn, use_elu)


def kernel(x, edge_index, W1l, b1l, W1r, b1r, att1, bias1, bn1_g, bn1_b,
           W2l, b2l, W2r, b2r, att2, bias2, bn2_g, bn2_b,
           W3l, b3l, W3r, b3r, att3, bias3):
    loop = jnp.arange(N, dtype=edge_index.dtype)
    src = jnp.concatenate([edge_index[0], loop])
    dst = jnp.concatenate([edge_index[1], loop])

    h1 = _gat_layer(x, src, dst, W1l, b1l, W1r, b1r, att1, bias1, 8,
                    bn1_g, bn1_b, True, True)
    h2 = _gat_layer(h1, src, dst, W2l, b2l, W2r, b2r, att2, bias2, 4,
                    bn2_g, bn2_b, True, True)

    w3l_p = jnp.pad(W3l, ((0, 0), (0, 14)))
    b3l_p = jnp.pad(b3l, (0, 14))
    w3r_p = jnp.pad(W3r, ((0, 0), (0, 14)))
    b3r_p = jnp.pad(b3r, (0, 14))
    att3_p = jnp.pad(att3, ((0, 0), (0, 14)))
    bias3_p = jnp.pad(bias3, (0, 14))
    zeros16 = jnp.zeros((16,), jnp.float32)
    h3 = _gat_layer(h2, src, dst, w3l_p, b3l_p, w3r_p, b3r_p, att3_p,
                    bias3_p, 1, zeros16, zeros16, False, False)
    return h3[:, :2]


# speedup vs baseline: 17.9572x; 2.3016x over previous
import functools

import jax
import jax.numpy as jnp
from jax import lax
from jax.experimental import pallas as pl
from jax.experimental.pallas import tpu as pltpu
from jax.experimental.pallas import tpu_sc as plsc

N = 50000
E_RAW = 800000
E_TOT = E_RAW + N
E_PAD = 851968

EB = 2048
NB = 2000

NW = 32
GCB = 512


def _proj_body(x_ref, w_ref, b_ref, o_ref):
    o_ref[...] = (
        jnp.dot(x_ref[...], w_ref[...], preferred_element_type=jnp.float32)
        + b_ref[...]
    )


def _proj(x, w, b):
    n, din = x.shape
    d = w.shape[1]
    grid = n // NB
    return pl.pallas_call(
        _proj_body,
        grid=(grid,),
        in_specs=[
            pl.BlockSpec((NB, din), lambda i: (i, 0)),
            pl.BlockSpec((din, d), lambda i: (0, 0)),
            pl.BlockSpec((1, d), lambda i: (0, 0)),
        ],
        out_specs=pl.BlockSpec((NB, d), lambda i: (i, 0)),
        out_shape=jax.ShapeDtypeStruct((n, d), jnp.float32),
    )(x, w, b.reshape(1, d))


def _alpha_body(gl_ref, gr_ref, att_ref, a_ref, m_ref, *, h):
    e = gl_ref[...] + gr_ref[...]
    e = jnp.where(e > 0, e, 0.2 * e)
    eb = e.shape[0]
    a = (e.reshape(eb, h, 16) * att_ref[...]).sum(-1)
    if h < 8:
        a = jnp.pad(a, ((0, 0), (0, 8 - h)))
    a_ref[...] = a
    m_ref[...] = jnp.max(a, axis=0, keepdims=True)[None]


def _alpha(gl, gr, att):
    h = att.shape[0]
    d = gl.shape[1]
    grid = E_PAD // EB
    return pl.pallas_call(
        functools.partial(_alpha_body, h=h),
        grid=(grid,),
        in_specs=[
            pl.BlockSpec((EB, d), lambda i: (i, 0)),
            pl.BlockSpec((EB, d), lambda i: (i, 0)),
            pl.BlockSpec((h, 16), lambda i: (0, 0)),
        ],
        out_specs=[
            pl.BlockSpec((EB, 8), lambda i: (i, 0)),
            pl.BlockSpec((1, 1, 8), lambda i: (i, 0, 0)),
        ],
        out_shape=[
            jax.ShapeDtypeStruct((E_PAD, 8), jnp.float32),
            jax.ShapeDtypeStruct((grid, 1, 8), jnp.float32),
        ],
    )(gl, gr, att)


def _msg_body(gl_ref, a_ref, k_ref, msg_ref, s_ref, *, h):
    s = jnp.exp(a_ref[...] - k_ref[...])
    sexp = jnp.repeat(s[:, :h], 16, axis=1)
    msg_ref[...] = gl_ref[...] * sexp
    s_ref[...] = jnp.pad(s, ((0, 0), (0, 8)))


def _msg(gl, alpha, kmax, h):
    d = gl.shape[1]
    grid = E_PAD // EB
    return pl.pallas_call(
        functools.partial(_msg_body, h=h),
        grid=(grid,),
        in_specs=[
            pl.BlockSpec((EB, d), lambda i: (i, 0)),
            pl.BlockSpec((EB, 8), lambda i: (i, 0)),
            pl.BlockSpec((1, 1), lambda i: (0, 0)),
        ],
        out_specs=[
            pl.BlockSpec((EB, d), lambda i: (i, 0)),
            pl.BlockSpec((EB, 16), lambda i: (i, 0)),
        ],
        out_shape=[
            jax.ShapeDtypeStruct((E_PAD, d), jnp.float32),
            jax.ShapeDtypeStruct((E_PAD, 16), jnp.float32),
        ],
    )(gl, alpha, kmax.reshape(1, 1))


def _final_body(acc_ref, den_ref, b_ref, g_ref, bb_ref, o_ref, *, h, bn, elu):
    den = den_ref[...][:, :h]
    dexp = jnp.repeat(den, 16, axis=1) + 1e-16
    o = acc_ref[...] / dexp + b_ref[...]
    if bn:
        o = o / jnp.sqrt(jnp.float32(1.0 + 1e-5)) * g_ref[...] + bb_ref[...]
    if elu:
        o = jnp.where(o > 0, o, jnp.exp(jnp.minimum(o, 0.0)) - 1.0)
    o_ref[...] = o


def _final(acc, den, bias, g, bb, h, bn, elu):
    d = acc.shape[1]
    grid = N // NB
    return pl.pallas_call(
        functools.partial(_final_body, h=h, bn=bn, elu=elu),
        grid=(grid,),
        in_specs=[
            pl.BlockSpec((NB, d), lambda i: (i, 0)),
            pl.BlockSpec((NB, 16), lambda i: (i, 0)),
            pl.BlockSpec((1, d), lambda i: (0, 0)),
            pl.BlockSpec((1, d), lambda i: (0, 0)),
            pl.BlockSpec((1, d), lambda i: (0, 0)),
        ],
        out_specs=pl.BlockSpec((NB, d), lambda i: (i, 0)),
        out_shape=jax.ShapeDtypeStruct((N, d), jnp.float32),
    )(acc, den, bias.reshape(1, d), g.reshape(1, d), bb.reshape(1, d))


_SC_MESH = plsc.VectorSubcoreMesh(core_axis_name="c", subcore_axis_name="s")


def _gather_rows(table, idx):
    d = table.shape[1]
    b_per_w = E_PAD // NW

    @functools.partial(
        pl.kernel,
        out_type=jax.ShapeDtypeStruct((E_PAD, d), jnp.float32),
        mesh=_SC_MESH,
        compiler_params=pltpu.CompilerParams(use_tc_tiling_on_sc=False),
        scratch_types=[
            pltpu.VMEM((GCB,), jnp.int32),
            pltpu.VMEM((GCB, d), jnp.float32),
            pltpu.SemaphoreType.DMA,
        ],
    )
    def k(table_hbm, idx_hbm, out_hbm, idx_v, rows_v, sem):
        wid = lax.axis_index("s") * 2 + lax.axis_index("c")
        base = wid * b_per_w

        def body(j, carry):
            off = base + j * GCB
            pltpu.sync_copy(idx_hbm.at[pl.ds(off, GCB)], idx_v)
            pltpu.async_copy(table_hbm.at[idx_v], rows_v, sem).wait()
            pltpu.sync_copy(rows_v, out_hbm.at[pl.ds(off, GCB)])
            return carry

        lax.fori_loop(0, b_per_w // GCB, body, 0)

    return k(table, idx)


def _scatter_add_rows(vals, idx, n):
    del n
    d = vals.shape[1]
    nchunk = 4 if d > 64 else 2
    ch = 50048 // nchunk
    acc_rows = ch + 16
    zpt = acc_rows // 16
    wpt = ch // 16
    ept = E_PAD // 16
    sb = 128 if d >= 128 else (256 if d == 64 else 512)
    zeros = jnp.zeros((acc_rows, d), jnp.float32)

    @functools.partial(
        pl.kernel,
        out_type=jax.ShapeDtypeStruct((50048, d), jnp.float32),
        mesh=_SC_MESH,
        compiler_params=pltpu.CompilerParams(use_tc_tiling_on_sc=False),
        scratch_types=[
            pltpu.VMEM_SHARED((acc_rows, d), jnp.float32),
            pltpu.VMEM((sb,), jnp.int32),
            pltpu.VMEM((sb,), jnp.int32),
            pltpu.VMEM((sb, d), jnp.float32),
        ],
    )
    def k(vals_hbm, dst_hbm, zeros_hbm, out_hbm, acc, dstv, lidxv, rowsv):
        cid = lax.axis_index("c")
        sid = lax.axis_index("s")
        for kk in range(nchunk // 2):
            base = (cid * (nchunk // 2) + kk) * ch
            pltpu.sync_copy(zeros_hbm.at[pl.ds(sid * zpt, zpt)],
                            acc.at[pl.ds(sid * zpt, zpt)])
            plsc.subcore_barrier()

            def body(j, carry):
                off = sid * ept + j * sb
                pltpu.sync_copy(dst_hbm.at[pl.ds(off, sb)], dstv)
                pltpu.sync_copy(vals_hbm.at[pl.ds(off, sb)], rowsv)

                def vb(i, c2):
                    v = dstv[pl.ds(i * 16, 16)]
                    m = (v >= base) & (v < base + ch)
                    lidxv[pl.ds(i * 16, 16)] = jnp.where(m, v - base, ch)
                    return c2

                lax.fori_loop(0, sb // 16, vb, 0)
                pltpu.sync_copy(rowsv, acc.at[lidxv], add=True)
                return carry

            lax.fori_loop(0, ept // sb, body, 0)
            plsc.subcore_barrier()
            pltpu.sync_copy(acc.at[pl.ds(sid * wpt, wpt)],
                            out_hbm.at[pl.ds(base + sid * wpt, wpt)])

    return k(vals, idx, zeros)


def _gat_layer(x, src, dst, wl, bl, wr, br, att, bias, h, bn_g, bn_b,
               use_bn, use_elu):
    d = wl.shape[1]
    xl = _proj(x, wl, bl)
    xr = _proj(x, wr, br)
    gl = _gather_rows(xl, src)
    gr = _gather_rows(xr, dst)
    alpha, bmax = _alpha(gl, gr, att)
    kmax = jnp.max(bmax)
    msg, spad = _msg(gl, alpha, kmax, h)
    acc = _scatter_add_rows(msg, dst, 50048)
    den = _scatter_add_rows(spad, dst, 50048)
    return _final(acc, den, bias, bn_g, bn_b, h, use_bn, use_elu)


def kernel(x, edge_index, W1l, b1l, W1r, b1r, att1, bias1, bn1_g, bn1_b,
           W2l, b2l, W2r, b2r, att2, bias2, bn2_g, bn2_b,
           W3l, b3l, W3r, b3r, att3, bias3):
    loop = jnp.arange(N, dtype=jnp.int32)
    srcpad = jnp.zeros((E_PAD - E_TOT,), jnp.int32)
    dstpad = jnp.full((E_PAD - E_TOT,), N, jnp.int32)
    src = jnp.concatenate([edge_index[0].astype(jnp.int32), loop, srcpad])
    dst = jnp.concatenate([edge_index[1].astype(jnp.int32), loop, dstpad])

    h1 = _gat_layer(x, src, dst, W1l, b1l, W1r, b1r, att1, bias1, 8,
                    bn1_g, bn1_b, True, True)
    h2 = _gat_layer(h1, src, dst, W2l, b2l, W2r, b2r, att2, bias2, 4,
                    bn2_g, bn2_b, True, True)

    w3l_p = jnp.pad(W3l, ((0, 0), (0, 14)))
    b3l_p = jnp.pad(b3l, (0, 14))
    w3r_p = jnp.pad(W3r, ((0, 0), (0, 14)))
    b3r_p = jnp.pad(b3r, (0, 14))
    att3_p = jnp.pad(att3, ((0, 0), (0, 14)))
    bias3_p = jnp.pad(bias3, (0, 14))
    zeros16 = jnp.zeros((16,), jnp.float32)
    h3 = _gat_layer(h2, src, dst, w3l_p, b3l_p, w3r_p, b3r_p, att3_p,
                    bias3_p, 1, zeros16, zeros16, False, False)
    return h3[:, :2]


# speedup vs baseline: 18.0717x; 1.0064x over previous
import functools

import jax
import jax.numpy as jnp
from jax import lax
from jax.experimental import pallas as pl
from jax.experimental.pallas import tpu as pltpu
from jax.experimental.pallas import tpu_sc as plsc

N = 50000
E_RAW = 800000
E_TOT = E_RAW + N
E_PAD = 851968

EB = 2048
NB = 2000

NW = 32
GCB = 512


def _proj_body(x_ref, w_ref, b_ref, o_ref):
    o_ref[...] = (
        jnp.dot(x_ref[...], w_ref[...], preferred_element_type=jnp.float32)
        + b_ref[...]
    )


def _proj(x, w, b):
    n, din = x.shape
    d = w.shape[1]
    grid = n // NB
    return pl.pallas_call(
        _proj_body,
        grid=(grid,),
        in_specs=[
            pl.BlockSpec((NB, din), lambda i: (i, 0)),
            pl.BlockSpec((din, d), lambda i: (0, 0)),
            pl.BlockSpec((1, d), lambda i: (0, 0)),
        ],
        out_specs=pl.BlockSpec((NB, d), lambda i: (i, 0)),
        out_shape=jax.ShapeDtypeStruct((n, d), jnp.float32),
    )(x, w, b.reshape(1, d))


def _alpha_body(gl_ref, gr_ref, att_ref, a_ref, m_ref, *, h):
    e = gl_ref[...] + gr_ref[...]
    e = jnp.where(e > 0, e, 0.2 * e)
    eb = e.shape[0]
    a = (e.reshape(eb, h, 16) * att_ref[...]).sum(-1)
    if h < 8:
        a = jnp.pad(a, ((0, 0), (0, 8 - h)))
    a_ref[...] = a
    m_ref[...] = jnp.max(a, axis=0, keepdims=True)[None]


def _alpha(gl, gr, att):
    h = att.shape[0]
    d = gl.shape[1]
    grid = E_PAD // EB
    return pl.pallas_call(
        functools.partial(_alpha_body, h=h),
        grid=(grid,),
        in_specs=[
            pl.BlockSpec((EB, d), lambda i: (i, 0)),
            pl.BlockSpec((EB, d), lambda i: (i, 0)),
            pl.BlockSpec((h, 16), lambda i: (0, 0)),
        ],
        out_specs=[
            pl.BlockSpec((EB, 8), lambda i: (i, 0)),
            pl.BlockSpec((1, 1, 8), lambda i: (i, 0, 0)),
        ],
        out_shape=[
            jax.ShapeDtypeStruct((E_PAD, 8), jnp.float32),
            jax.ShapeDtypeStruct((grid, 1, 8), jnp.float32),
        ],
    )(gl, gr, att)


def _msg_body(gl_ref, a_ref, k_ref, msg_ref, s_ref, *, h):
    s = jnp.exp(a_ref[...] - k_ref[...])
    sexp = jnp.repeat(s[:, :h], 16, axis=1)
    msg_ref[...] = gl_ref[...] * sexp
    s_ref[...] = jnp.pad(s, ((0, 0), (0, 8)))


def _msg(gl, alpha, kmax, h):
    d = gl.shape[1]
    grid = E_PAD // EB
    return pl.pallas_call(
        functools.partial(_msg_body, h=h),
        grid=(grid,),
        in_specs=[
            pl.BlockSpec((EB, d), lambda i: (i, 0)),
            pl.BlockSpec((EB, 8), lambda i: (i, 0)),
            pl.BlockSpec((1, 1), lambda i: (0, 0)),
        ],
        out_specs=[
            pl.BlockSpec((EB, d), lambda i: (i, 0)),
            pl.BlockSpec((EB, 16), lambda i: (i, 0)),
        ],
        out_shape=[
            jax.ShapeDtypeStruct((E_PAD, d), jnp.float32),
            jax.ShapeDtypeStruct((E_PAD, 16), jnp.float32),
        ],
    )(gl, alpha, kmax.reshape(1, 1))


def _final_body(acc_ref, den_ref, b_ref, g_ref, bb_ref, o_ref, *, h, bn, elu):
    den = den_ref[...][:, :h]
    dexp = jnp.repeat(den, 16, axis=1) + 1e-35
    o = acc_ref[...] / dexp + b_ref[...]
    if bn:
        o = o / jnp.sqrt(jnp.float32(1.0 + 1e-5)) * g_ref[...] + bb_ref[...]
    if elu:
        o = jnp.where(o > 0, o, jnp.exp(jnp.minimum(o, 0.0)) - 1.0)
    o_ref[...] = o


def _final(acc, den, bias, g, bb, h, bn, elu):
    d = acc.shape[1]
    grid = N // NB
    return pl.pallas_call(
        functools.partial(_final_body, h=h, bn=bn, elu=elu),
        grid=(grid,),
        in_specs=[
            pl.BlockSpec((NB, d), lambda i: (i, 0)),
            pl.BlockSpec((NB, 16), lambda i: (i, 0)),
            pl.BlockSpec((1, d), lambda i: (0, 0)),
            pl.BlockSpec((1, d), lambda i: (0, 0)),
            pl.BlockSpec((1, d), lambda i: (0, 0)),
        ],
        out_specs=pl.BlockSpec((NB, d), lambda i: (i, 0)),
        out_shape=jax.ShapeDtypeStruct((N, d), jnp.float32),
    )(acc, den, bias.reshape(1, d), g.reshape(1, d), bb.reshape(1, d))


_SC_MESH = plsc.VectorSubcoreMesh(core_axis_name="c", subcore_axis_name="s")


def _gather_rows(table, idx):
    d = table.shape[1]
    b_per_w = E_PAD // NW
    gcb = 416 if d >= 128 else (832 if d == 64 else 1024)

    @functools.partial(
        pl.kernel,
        out_type=jax.ShapeDtypeStruct((E_PAD, d), jnp.float32),
        mesh=_SC_MESH,
        compiler_params=pltpu.CompilerParams(use_tc_tiling_on_sc=False),
        scratch_types=[
            pltpu.VMEM((gcb,), jnp.int32),
            pltpu.VMEM((gcb,), jnp.int32),
            pltpu.VMEM((gcb, d), jnp.float32),
            pltpu.VMEM((gcb, d), jnp.float32),
            pltpu.SemaphoreType.DMA,
            pltpu.SemaphoreType.DMA,
            pltpu.SemaphoreType.DMA,
            pltpu.SemaphoreType.DMA,
            pltpu.SemaphoreType.DMA,
            pltpu.SemaphoreType.DMA,
        ],
    )
    def k(table_hbm, idx_hbm, out_hbm, idx0, idx1, rows0, rows1,
          si0, si1, sg0, sg1, sw0, sw1):
        wid = lax.axis_index("s") * 2 + lax.axis_index("c")
        base = wid * b_per_w

        def body(i, carry):
            o0 = base + (2 * i) * gcb
            o1 = o0 + gcb
            hi0 = pltpu.async_copy(idx_hbm.at[pl.ds(o0, gcb)], idx0, si0)
            hi1 = pltpu.async_copy(idx_hbm.at[pl.ds(o1, gcb)], idx1, si1)
            hi0.wait()
            g0 = pltpu.async_copy(table_hbm.at[idx0], rows0, sg0)
            hi1.wait()
            g0.wait()
            w0 = pltpu.async_copy(rows0, out_hbm.at[pl.ds(o0, gcb)], sw0)
            g1 = pltpu.async_copy(table_hbm.at[idx1], rows1, sg1)
            g1.wait()
            w1 = pltpu.async_copy(rows1, out_hbm.at[pl.ds(o1, gcb)], sw1)
            w0.wait()
            w1.wait()
            return carry

        lax.fori_loop(0, b_per_w // (2 * gcb), body, 0)

    return k(table, idx)


def _scatter_add_rows(vals, idx, n):
    del n
    d = vals.shape[1]
    nchunk = 4 if d > 64 else 2
    ch = 50048 // nchunk
    acc_rows = ch + 16
    zpt = acc_rows // 16
    wpt = ch // 16
    ept = E_PAD // 16
    sb = 128 if d >= 128 else (256 if d == 64 else 512)
    zeros = jnp.zeros((acc_rows, d), jnp.float32)

    @functools.partial(
        pl.kernel,
        out_type=jax.ShapeDtypeStruct((50048, d), jnp.float32),
        mesh=_SC_MESH,
        compiler_params=pltpu.CompilerParams(use_tc_tiling_on_sc=False),
        scratch_types=[
            pltpu.VMEM_SHARED((acc_rows, d), jnp.float32),
            pltpu.VMEM((sb,), jnp.int32),
            pltpu.VMEM((sb,), jnp.int32),
            pltpu.VMEM((sb, d), jnp.float32),
        ],
    )
    def k(vals_hbm, dst_hbm, zeros_hbm, out_hbm, acc, dstv, lidxv, rowsv):
        cid = lax.axis_index("c")
        sid = lax.axis_index("s")
        for kk in range(nchunk // 2):
            base = (cid * (nchunk // 2) + kk) * ch
            pltpu.sync_copy(zeros_hbm.at[pl.ds(sid * zpt, zpt)],
                            acc.at[pl.ds(sid * zpt, zpt)])
            plsc.subcore_barrier()

            def body(j, carry):
                off = sid * ept + j * sb
                pltpu.sync_copy(dst_hbm.at[pl.ds(off, sb)], dstv)
                pltpu.sync_copy(vals_hbm.at[pl.ds(off, sb)], rowsv)

                def vb(i, c2):
                    v = dstv[pl.ds(i * 16, 16)]
                    m = (v >= base) & (v < base + ch)
                    lidxv[pl.ds(i * 16, 16)] = jnp.where(m, v - base, ch)
                    return c2

                lax.fori_loop(0, sb // 16, vb, 0)
                pltpu.sync_copy(rowsv, acc.at[lidxv], add=True)
                return carry

            lax.fori_loop(0, ept // sb, body, 0)
            plsc.subcore_barrier()
            pltpu.sync_copy(acc.at[pl.ds(sid * wpt, wpt)],
                            out_hbm.at[pl.ds(base + sid * wpt, wpt)])

    return k(vals, idx, zeros)


def _gat_layer(x, src, dst, wl, bl, wr, br, att, bias, h, bn_g, bn_b,
               use_bn, use_elu):
    d = wl.shape[1]
    xl = _proj(x, wl, bl)
    xr = _proj(x, wr, br)
    gl = _gather_rows(xl, src)
    gr = _gather_rows(xr, dst)
    alpha, bmax = _alpha(gl, gr, att)
    kmax = jnp.max(bmax)
    msg, spad = _msg(gl, alpha, kmax, h)
    acc = _scatter_add_rows(msg, dst, 50048)
    den = _scatter_add_rows(spad, dst, 50048)
    return _final(acc, den, bias, bn_g, bn_b, h, use_bn, use_elu)


def kernel(x, edge_index, W1l, b1l, W1r, b1r, att1, bias1, bn1_g, bn1_b,
           W2l, b2l, W2r, b2r, att2, bias2, bn2_g, bn2_b,
           W3l, b3l, W3r, b3r, att3, bias3):
    loop = jnp.arange(N, dtype=jnp.int32)
    srcpad = jnp.zeros((E_PAD - E_TOT,), jnp.int32)
    dstpad = jnp.full((E_PAD - E_TOT,), N, jnp.int32)
    src = jnp.concatenate([edge_index[0].astype(jnp.int32), loop, srcpad])
    dst = jnp.concatenate([edge_index[1].astype(jnp.int32), loop, dstpad])

    h1 = _gat_layer(x, src, dst, W1l, b1l, W1r, b1r, att1, bias1, 8,
                    bn1_g, bn1_b, True, True)
    h2 = _gat_layer(h1, src, dst, W2l, b2l, W2r, b2r, att2, bias2, 4,
                    bn2_g, bn2_b, True, True)

    w3l_p = jnp.pad(W3l, ((0, 0), (0, 14)))
    b3l_p = jnp.pad(b3l, (0, 14))
    w3r_p = jnp.pad(W3r, ((0, 0), (0, 14)))
    b3r_p = jnp.pad(b3r, (0, 14))
    att3_p = jnp.pad(att3, ((0, 0), (0, 14)))
    bias3_p = jnp.pad(bias3, (0, 14))
    zeros16 = jnp.zeros((16,), jnp.float32)
    h3 = _gat_layer(h2, src, dst, w3l_p, b3l_p, w3r_p, b3r_p, att3_p,
                    bias3_p, 1, zeros16, zeros16, False, False)
    return h3[:, :2]


# speedup vs baseline: 18.1008x; 1.0016x over previous
import functools

import jax
import jax.numpy as jnp
from jax import lax
from jax.experimental import pallas as pl
from jax.experimental.pallas import tpu as pltpu
from jax.experimental.pallas import tpu_sc as plsc

N = 50000
E_RAW = 800000
E_TOT = E_RAW + N
E_PAD = 851968

EB = 2048
NB = 2000

NW = 32
GCB = 512


def _proj_body(x_ref, w_ref, b_ref, o_ref):
    o_ref[...] = (
        jnp.dot(x_ref[...], w_ref[...], preferred_element_type=jnp.float32)
        + b_ref[...]
    )


def _proj(x, w, b):
    n, din = x.shape
    d = w.shape[1]
    grid = n // NB
    return pl.pallas_call(
        _proj_body,
        grid=(grid,),
        in_specs=[
            pl.BlockSpec((NB, din), lambda i: (i, 0)),
            pl.BlockSpec((din, d), lambda i: (0, 0)),
            pl.BlockSpec((1, d), lambda i: (0, 0)),
        ],
        out_specs=pl.BlockSpec((NB, d), lambda i: (i, 0)),
        out_shape=jax.ShapeDtypeStruct((n, d), jnp.float32),
    )(x, w, b.reshape(1, d))


def _alpha_body(gl_ref, gr_ref, att_ref, a_ref, m_ref, *, h):
    e = gl_ref[...] + gr_ref[...]
    e = jnp.where(e > 0, e, 0.2 * e)
    eb = e.shape[0]
    a = (e.reshape(eb, h, 16) * att_ref[...]).sum(-1)
    if h < 8:
        a = jnp.pad(a, ((0, 0), (0, 8 - h)))
    a_ref[...] = a
    m_ref[...] = jnp.max(a, axis=0, keepdims=True)[None]


def _alpha(gl, gr, att):
    h = att.shape[0]
    d = gl.shape[1]
    grid = E_PAD // EB
    return pl.pallas_call(
        functools.partial(_alpha_body, h=h),
        grid=(grid,),
        in_specs=[
            pl.BlockSpec((EB, d), lambda i: (i, 0)),
            pl.BlockSpec((EB, d), lambda i: (i, 0)),
            pl.BlockSpec((h, 16), lambda i: (0, 0)),
        ],
        out_specs=[
            pl.BlockSpec((EB, 8), lambda i: (i, 0)),
            pl.BlockSpec((1, 1, 8), lambda i: (i, 0, 0)),
        ],
        out_shape=[
            jax.ShapeDtypeStruct((E_PAD, 8), jnp.float32),
            jax.ShapeDtypeStruct((grid, 1, 8), jnp.float32),
        ],
    )(gl, gr, att)


def _msg_body(gl_ref, a_ref, k_ref, msg_ref, s_ref, *, h):
    s = jnp.exp(a_ref[...] - k_ref[...])
    sexp = jnp.repeat(s[:, :h], 16, axis=1)
    msg_ref[...] = gl_ref[...] * sexp
    s_ref[...] = jnp.pad(s, ((0, 0), (0, 8)))


def _msg(gl, alpha, kmax, h):
    d = gl.shape[1]
    grid = E_PAD // EB
    return pl.pallas_call(
        functools.partial(_msg_body, h=h),
        grid=(grid,),
        in_specs=[
            pl.BlockSpec((EB, d), lambda i: (i, 0)),
            pl.BlockSpec((EB, 8), lambda i: (i, 0)),
            pl.BlockSpec((1, 1), lambda i: (0, 0)),
        ],
        out_specs=[
            pl.BlockSpec((EB, d), lambda i: (i, 0)),
            pl.BlockSpec((EB, 16), lambda i: (i, 0)),
        ],
        out_shape=[
            jax.ShapeDtypeStruct((E_PAD, d), jnp.float32),
            jax.ShapeDtypeStruct((E_PAD, 16), jnp.float32),
        ],
    )(gl, alpha, kmax.reshape(1, 1))


def _final_body(acc_ref, den_ref, b_ref, g_ref, bb_ref, o_ref, *, h, bn, elu):
    den = den_ref[...][:, :h]
    dexp = jnp.repeat(den, 16, axis=1) + 1e-35
    o = acc_ref[...] / dexp + b_ref[...]
    if bn:
        o = o / jnp.sqrt(jnp.float32(1.0 + 1e-5)) * g_ref[...] + bb_ref[...]
    if elu:
        o = jnp.where(o > 0, o, jnp.exp(jnp.minimum(o, 0.0)) - 1.0)
    o_ref[...] = o


def _final(acc, den, bias, g, bb, h, bn, elu):
    d = acc.shape[1]
    grid = N // NB
    return pl.pallas_call(
        functools.partial(_final_body, h=h, bn=bn, elu=elu),
        grid=(grid,),
        in_specs=[
            pl.BlockSpec((NB, d), lambda i: (i, 0)),
            pl.BlockSpec((NB, 16), lambda i: (i, 0)),
            pl.BlockSpec((1, d), lambda i: (0, 0)),
            pl.BlockSpec((1, d), lambda i: (0, 0)),
            pl.BlockSpec((1, d), lambda i: (0, 0)),
        ],
        out_specs=pl.BlockSpec((NB, d), lambda i: (i, 0)),
        out_shape=jax.ShapeDtypeStruct((N, d), jnp.float32),
    )(acc, den, bias.reshape(1, d), g.reshape(1, d), bb.reshape(1, d))


_SC_MESH = plsc.VectorSubcoreMesh(core_axis_name="c", subcore_axis_name="s")


def _gather_rows(table, idx):
    d = table.shape[1]
    b_per_w = E_PAD // NW
    gcb = 416 if d >= 128 else (832 if d == 64 else 1024)

    @functools.partial(
        pl.kernel,
        out_type=jax.ShapeDtypeStruct((E_PAD, d), jnp.float32),
        mesh=_SC_MESH,
        compiler_params=pltpu.CompilerParams(use_tc_tiling_on_sc=False),
        scratch_types=[
            pltpu.VMEM((gcb,), jnp.int32),
            pltpu.VMEM((gcb,), jnp.int32),
            pltpu.VMEM((gcb, d), jnp.float32),
            pltpu.VMEM((gcb, d), jnp.float32),
            pltpu.SemaphoreType.DMA,
            pltpu.SemaphoreType.DMA,
            pltpu.SemaphoreType.DMA,
            pltpu.SemaphoreType.DMA,
            pltpu.SemaphoreType.DMA,
            pltpu.SemaphoreType.DMA,
        ],
    )
    def k(table_hbm, idx_hbm, out_hbm, idx0, idx1, rows0, rows1,
          si0, si1, sg0, sg1, sw0, sw1):
        wid = lax.axis_index("s") * 2 + lax.axis_index("c")
        base = wid * b_per_w

        def body(i, carry):
            o0 = base + (2 * i) * gcb
            o1 = o0 + gcb
            hi0 = pltpu.async_copy(idx_hbm.at[pl.ds(o0, gcb)], idx0, si0)
            hi1 = pltpu.async_copy(idx_hbm.at[pl.ds(o1, gcb)], idx1, si1)
            hi0.wait()
            g0 = pltpu.async_copy(table_hbm.at[idx0], rows0, sg0)
            hi1.wait()
            g1 = pltpu.async_copy(table_hbm.at[idx1], rows1, sg1)
            g0.wait()
            w0 = pltpu.async_copy(rows0, out_hbm.at[pl.ds(o0, gcb)], sw0)
            g1.wait()
            w1 = pltpu.async_copy(rows1, out_hbm.at[pl.ds(o1, gcb)], sw1)
            w0.wait()
            w1.wait()
            return carry

        lax.fori_loop(0, b_per_w // (2 * gcb), body, 0)

    return k(table, idx)


def _scatter_add_rows(vals, idx, n):
    del n
    d = vals.shape[1]
    nchunk = 4 if d > 64 else 2
    ch = 50048 // nchunk
    acc_rows = ch + 16
    zpt = acc_rows // 16
    wpt = ch // 16
    ept = E_PAD // 16
    sb = 128 if d >= 128 else (256 if d == 64 else 512)
    zeros = jnp.zeros((acc_rows, d), jnp.float32)

    @functools.partial(
        pl.kernel,
        out_type=jax.ShapeDtypeStruct((50048, d), jnp.float32),
        mesh=_SC_MESH,
        compiler_params=pltpu.CompilerParams(use_tc_tiling_on_sc=False),
        scratch_types=[
            pltpu.VMEM_SHARED((acc_rows, d), jnp.float32),
            pltpu.VMEM((sb,), jnp.int32),
            pltpu.VMEM((sb,), jnp.int32),
            pltpu.VMEM((sb, d), jnp.float32),
        ],
    )
    def k(vals_hbm, dst_hbm, zeros_hbm, out_hbm, acc, dstv, lidxv, rowsv):
        cid = lax.axis_index("c")
        sid = lax.axis_index("s")
        for kk in range(nchunk // 2):
            base = (cid * (nchunk // 2) + kk) * ch
            pltpu.sync_copy(zeros_hbm.at[pl.ds(sid * zpt, zpt)],
                            acc.at[pl.ds(sid * zpt, zpt)])
            plsc.subcore_barrier()

            def body(j, carry):
                off = sid * ept + j * sb
                pltpu.sync_copy(dst_hbm.at[pl.ds(off, sb)], dstv)
                pltpu.sync_copy(vals_hbm.at[pl.ds(off, sb)], rowsv)

                def vb(i, c2):
                    v = dstv[pl.ds(i * 16, 16)]
                    m = (v >= base) & (v < base + ch)
                    lidxv[pl.ds(i * 16, 16)] = jnp.where(m, v - base, ch)
                    return c2

                lax.fori_loop(0, sb // 16, vb, 0)
                pltpu.sync_copy(rowsv, acc.at[lidxv], add=True)
                return carry

            lax.fori_loop(0, ept // sb, body, 0)
            plsc.subcore_barrier()
            pltpu.sync_copy(acc.at[pl.ds(sid * wpt, wpt)],
                            out_hbm.at[pl.ds(base + sid * wpt, wpt)])

    return k(vals, idx, zeros)


def _gat_layer(x, src, dst, wl, bl, wr, br, att, bias, h, bn_g, bn_b,
               use_bn, use_elu):
    d = wl.shape[1]
    xl = _proj(x, wl, bl)
    xr = _proj(x, wr, br)
    gl = _gather_rows(xl, src)
    gr = _gather_rows(xr, dst)
    alpha, bmax = _alpha(gl, gr, att)
    kmax = jnp.max(bmax)
    msg, spad = _msg(gl, alpha, kmax, h)
    acc = _scatter_add_rows(msg, dst, 50048)
    den = _scatter_add_rows(spad, dst, 50048)
    return _final(acc, den, bias, bn_g, bn_b, h, use_bn, use_elu)


def kernel(x, edge_index, W1l, b1l, W1r, b1r, att1, bias1, bn1_g, bn1_b,
           W2l, b2l, W2r, b2r, att2, bias2, bn2_g, bn2_b,
           W3l, b3l, W3r, b3r, att3, bias3):
    loop = jnp.arange(N, dtype=jnp.int32)
    srcpad = jnp.zeros((E_PAD - E_TOT,), jnp.int32)
    dstpad = jnp.full((E_PAD - E_TOT,), N, jnp.int32)
    src = jnp.concatenate([edge_index[0].astype(jnp.int32), loop, srcpad])
    dst = jnp.concatenate([edge_index[1].astype(jnp.int32), loop, dstpad])

    h1 = _gat_layer(x, src, dst, W1l, b1l, W1r, b1r, att1, bias1, 8,
                    bn1_g, bn1_b, True, True)
    h2 = _gat_layer(h1, src, dst, W2l, b2l, W2r, b2r, att2, bias2, 4,
                    bn2_g, bn2_b, True, True)

    w3l_p = jnp.pad(W3l, ((0, 0), (0, 14)))
    b3l_p = jnp.pad(b3l, (0, 14))
    w3r_p = jnp.pad(W3r, ((0, 0), (0, 14)))
    b3r_p = jnp.pad(b3r, (0, 14))
    att3_p = jnp.pad(att3, ((0, 0), (0, 14)))
    bias3_p = jnp.pad(bias3, (0, 14))
    zeros16 = jnp.zeros((16,), jnp.float32)
    h3 = _gat_layer(h2, src, dst, w3l_p, b3l_p, w3r_p, b3r_p, att3_p,
                    bias3_p, 1, zeros16, zeros16, False, False)
    return h3[:, :2]
